# baseline (speedup 1.0000x reference)
import functools

import jax
import jax.numpy as jnp
from jax import lax
from jax.experimental import pallas as pl
from jax.experimental.pallas import tpu as pltpu
from jax.experimental.pallas import tpu_sc as plsc

_NV = 33
_NV3 = _NV ** 3
_IMG = 512
_BATCH = 4
_NC = 2
_NS = 16
_NW = _NC * _NS
_WORKERS_PER_IMG = _NW // _BATCH
_PX_PER_IMG = _IMG * _IMG
_PX_PER_W = _PX_PER_IMG // _WORKERS_PER_IMG
_CHUNK = 4096
_N_CHUNKS = _PX_PER_W // _CHUNK
_VERT_PAD = 104
_RCP_PAD = 96
_GSTRIDE = _NV
_BSTRIDE = _NV * _NV
_CSTRIDE = _NV ** 3
_PLUT = 3 * _CSTRIDE + 5


def _locate(vert_v, rcp_v, x, c):
    lo = jnp.zeros((16,), jnp.int32)
    alo = jnp.zeros((16,), jnp.float32)
    for s in (16, 8, 4, 2, 1):
        m = lo + s
        v = plsc.load_gather(vert_v, [m + c * _NV])
        take = x >= v
        lo = jnp.where(take, m, lo)
        alo = jnp.where(take, v, alo)
    rcp = plsc.load_gather(rcp_v, [lo + c * 32])
    f = (x - alo) * rcp
    return lo, f


def _ailut_body(imgs_hbm, luts_hbm, verts_hbm, rcps_hbm, out_hbm,
                lut_v, vert_v, rcp_v, r_v, g_v, b_v):
    wid = lax.axis_index("s") * _NC + lax.axis_index("c")
    img = wid // _WORKERS_PER_IMG
    slot = wid % _WORKERS_PER_IMG

    pltpu.sync_copy(luts_hbm.at[pl.ds(img * _PLUT, _PLUT)], lut_v)
    pltpu.sync_copy(verts_hbm.at[pl.ds(img * _VERT_PAD, _VERT_PAD)], vert_v)
    pltpu.sync_copy(rcps_hbm.at[pl.ds(img * _RCP_PAD, _RCP_PAD)], rcp_v)

    img_base = img * 3 * _PX_PER_IMG
    base_px = slot * _PX_PER_W

    def group(i, carry):
        row = i >> 5
        p = (i & 31) * 16
        r = r_v[row, pl.ds(p, 16)]
        g = g_v[row, pl.ds(p, 16)]
        b = b_v[row, pl.ds(p, 16)]
        rid, rf = _locate(vert_v, rcp_v, r, 0)
        gid, gf = _locate(vert_v, rcp_v, g, 1)
        bid, bf = _locate(vert_v, rcp_v, b, 2)
        base3 = bid * _BSTRIDE + gid * _GSTRIDE + rid
        wr0 = 1.0 - rf
        w00 = (1.0 - bf) * (1.0 - gf)
        w01 = (1.0 - bf) * gf
        w10 = bf * (1.0 - gf)
        w11 = bf * gf
        cw = (
            (0, w00 * wr0), (1, w00 * rf),
            (_GSTRIDE, w01 * wr0), (_GSTRIDE + 1, w01 * rf),
            (_BSTRIDE, w10 * wr0), (_BSTRIDE + 1, w10 * rf),
            (_BSTRIDE + _GSTRIDE, w11 * wr0), (_BSTRIDE + _GSTRIDE + 1, w11 * rf),
        )
        outs = []
        for c in range(3):
            acc = jnp.zeros((16,), jnp.float32)
            for off, w in cw:
                v = plsc.load_gather(lut_v, [base3 + (c * _CSTRIDE + off)])
                acc = acc + w * v
            outs.append(acc)
        r_v[row, pl.ds(p, 16)] = outs[0]
        g_v[row, pl.ds(p, 16)] = outs[1]
        b_v[row, pl.ds(p, 16)] = outs[2]
        return carry

    rows_per_chunk = _CHUNK // _IMG
    for chunk in range(_N_CHUNKS):
        r0 = slot * (_PX_PER_W // _IMG) + chunk * rows_per_chunk
        pltpu.sync_copy(imgs_hbm.at[img, 0, pl.ds(r0, rows_per_chunk), :], r_v)
        pltpu.sync_copy(imgs_hbm.at[img, 1, pl.ds(r0, rows_per_chunk), :], g_v)
        pltpu.sync_copy(imgs_hbm.at[img, 2, pl.ds(r0, rows_per_chunk), :], b_v)
        lax.fori_loop(0, _CHUNK // 16, group, 0)
        pltpu.sync_copy(r_v, out_hbm.at[img, 0, pl.ds(r0, rows_per_chunk), :])
        pltpu.sync_copy(g_v, out_hbm.at[img, 1, pl.ds(r0, rows_per_chunk), :])
        pltpu.sync_copy(b_v, out_hbm.at[img, 2, pl.ds(r0, rows_per_chunk), :])


def _ailut_sc(imgs_flat, luts_packed, verts_pad, rcps_flat):
    mesh = plsc.VectorSubcoreMesh(core_axis_name="c", subcore_axis_name="s")
    run = functools.partial(
        pl.kernel,
        mesh=mesh,
        compiler_params=pltpu.CompilerParams(needs_layout_passes=False),
        out_type=jax.ShapeDtypeStruct((_BATCH, 3, _IMG, _IMG), jnp.float32),
        scratch_types=[
            pltpu.VMEM((_PLUT,), jnp.float32),
            pltpu.VMEM((_VERT_PAD,), jnp.float32),
            pltpu.VMEM((_RCP_PAD,), jnp.float32),
            pltpu.VMEM((_CHUNK // _IMG, _IMG), jnp.float32),
            pltpu.VMEM((_CHUNK // _IMG, _IMG), jnp.float32),
            pltpu.VMEM((_CHUNK // _IMG, _IMG), jnp.float32),
        ],
    )(_ailut_body)
    return run(imgs_flat, luts_packed, verts_pad, rcps_flat)


def _conv(x, w, b, stride):
    y = lax.conv_general_dilated(x, w, (stride, stride),
                                 padding=((1, 1), (1, 1)),
                                 dimension_numbers=('NCHW', 'OIHW', 'NCHW'))
    return y + b[None, :, None, None]


def _inorm(x, g, b, eps=1e-5):
    m = x.mean(axis=(2, 3), keepdims=True)
    v = (x * x).mean(axis=(2, 3), keepdims=True) - m * m
    return g[None, :, None, None] * (x - m) * lax.rsqrt(v + eps) + b[None, :, None, None]


def _resize_half(x):
    def down(v):
        a = v[..., 0::2, :]
        b = v[..., 1::2, :]
        bp = jnp.concatenate([jnp.zeros_like(b[..., :1, :]), b[..., :-1, :]],
                             axis=-2)
        an = jnp.concatenate([a[..., 1:, :], jnp.zeros_like(a[..., :1, :])],
                             axis=-2)
        y = 0.375 * (a + b) + 0.125 * (bp + an)
        scale = jnp.ones((y.shape[-2],), v.dtype)
        scale = scale.at[0].set(1 / 0.875).at[-1].set(1 / 0.875)
        return y * scale[:, None]
    y = down(x)
    return down(y.swapaxes(-1, -2)).swapaxes(-1, -2)


def kernel(imgs, conv_w0, conv_b0, conv_w1, conv_b1, conv_w2, conv_b2,
           conv_w3, conv_b3, conv_w4, conv_b4, in_g0, in_b0, in_g1, in_b1,
           in_g2, in_b2, in_g3, in_b3, Wg, bg, Wl, Wa, ba):
    b = imgs.shape[0]
    x = _resize_half(imgs)
    convs = ((conv_w0, conv_b0), (conv_w1, conv_b1), (conv_w2, conv_b2),
             (conv_w3, conv_b3), (conv_w4, conv_b4))
    norms = ((in_g0, in_b0), (in_g1, in_b1), (in_g2, in_b2), (in_g3, in_b3))
    for i in range(5):
        x = _conv(x, convs[i][0], convs[i][1], 2)
        x = jnp.maximum(x, 0.2 * x)
        if i < 4:
            x = _inorm(x, norms[i][0], norms[i][1])
    bb, cc, hh, ww = x.shape
    x = x.reshape(bb, cc, 2, hh // 2, 2, ww // 2).mean(axis=(3, 5))
    codes = x.reshape(bb, -1)

    weights = codes @ Wg.T + bg
    luts = weights @ Wl.T
    intervals = (codes @ Wa.T + ba).reshape(b, 3, _NV - 1)
    intervals = jax.nn.softmax(intervals, axis=-1)
    vertices = jnp.pad(jnp.cumsum(intervals, axis=-1), ((0, 0), (0, 0), (1, 0)))

    luts_packed = jnp.pad(luts, ((0, 0), (0, 5))).reshape(-1)
    verts_pad = jnp.pad(vertices.reshape(b, 3 * _NV),
                        ((0, 0), (0, _VERT_PAD - 3 * _NV))).reshape(-1)
    dv = vertices[..., 1:] - vertices[..., :-1]
    rcps_flat = (1.0 / (dv + 1e-10)).reshape(-1)
    return _ailut_sc(imgs, luts_packed, verts_pad, rcps_flat)


# speedup vs baseline: 1.9703x; 1.9703x over previous
import functools

import jax
import jax.numpy as jnp
from jax import lax
from jax.experimental import pallas as pl
from jax.experimental.pallas import tpu as pltpu
from jax.experimental.pallas import tpu_sc as plsc

_NV = 33
_NV3 = _NV ** 3
_IMG = 512
_BATCH = 4
_NC = 2
_NS = 16
_NW = _NC * _NS
_WORKERS_PER_IMG = _NW // _BATCH
_PX_PER_IMG = _IMG * _IMG
_PX_PER_W = _PX_PER_IMG // _WORKERS_PER_IMG
_CHUNK = 4096
_N_CHUNKS = _PX_PER_W // _CHUNK
_VERT_PAD = 104
_RCP_PAD = 96
_GSTRIDE = _NV
_BSTRIDE = _NV * _NV
_CSTRIDE = _NV ** 3
_PLUT = 3 * _CSTRIDE + 5


def _locate(vert_v, rcp_v, x, c):
    lo = jnp.zeros((16,), jnp.int32)
    alo = jnp.zeros((16,), jnp.float32)
    for s in (16, 8, 4, 2, 1):
        m = lo + s
        v = plsc.load_gather(vert_v, [m + c * _NV])
        take = x >= v
        lo = jnp.where(take, m, lo)
        alo = jnp.where(take, v, alo)
    rcp = plsc.load_gather(rcp_v, [lo + c * 32])
    f = (x - alo) * rcp
    return lo, f


def _ailut_body(imgs_hbm, luts_hbm, verts_hbm, rcps_hbm, out_hbm,
                lut_v, vert_v, rcp_v, r_v, g_v, b_v):
    wid = lax.axis_index("s") * _NC + lax.axis_index("c")
    img = wid // _WORKERS_PER_IMG
    slot = wid % _WORKERS_PER_IMG

    pltpu.sync_copy(luts_hbm.at[pl.ds(img * _PLUT, _PLUT)], lut_v)
    pltpu.sync_copy(verts_hbm.at[pl.ds(img * _VERT_PAD, _VERT_PAD)], vert_v)
    pltpu.sync_copy(rcps_hbm.at[pl.ds(img * _RCP_PAD, _RCP_PAD)], rcp_v)

    img_base = img * 3 * _PX_PER_IMG
    base_px = slot * _PX_PER_W

    def group(i, carry):
        row = i >> 5
        p = (i & 31) * 16
        r = r_v[row, pl.ds(p, 16)]
        g = g_v[row, pl.ds(p, 16)]
        b = b_v[row, pl.ds(p, 16)]
        rid, rf = _locate(vert_v, rcp_v, r, 0)
        gid, gf = _locate(vert_v, rcp_v, g, 1)
        bid, bf = _locate(vert_v, rcp_v, b, 2)
        base3 = bid * _BSTRIDE + gid * _GSTRIDE + rid
        wr0 = 1.0 - rf
        w00 = (1.0 - bf) * (1.0 - gf)
        w01 = (1.0 - bf) * gf
        w10 = bf * (1.0 - gf)
        w11 = bf * gf
        cw = (
            (0, w00 * wr0), (1, w00 * rf),
            (_GSTRIDE, w01 * wr0), (_GSTRIDE + 1, w01 * rf),
            (_BSTRIDE, w10 * wr0), (_BSTRIDE + 1, w10 * rf),
            (_BSTRIDE + _GSTRIDE, w11 * wr0), (_BSTRIDE + _GSTRIDE + 1, w11 * rf),
        )
        outs = []
        for c in range(3):
            acc = jnp.zeros((16,), jnp.float32)
            for off, w in cw:
                v = plsc.load_gather(lut_v, [base3 + (c * _CSTRIDE + off)])
                acc = acc + w * v
            outs.append(acc)
        r_v[row, pl.ds(p, 16)] = outs[0]
        g_v[row, pl.ds(p, 16)] = outs[1]
        b_v[row, pl.ds(p, 16)] = outs[2]
        return carry

    rows_per_chunk = _CHUNK // _IMG
    for chunk in range(_N_CHUNKS):
        r0 = slot * (_PX_PER_W // _IMG) + chunk * rows_per_chunk
        pltpu.sync_copy(imgs_hbm.at[img, 0, pl.ds(r0, rows_per_chunk), :], r_v)
        pltpu.sync_copy(imgs_hbm.at[img, 1, pl.ds(r0, rows_per_chunk), :], g_v)
        pltpu.sync_copy(imgs_hbm.at[img, 2, pl.ds(r0, rows_per_chunk), :], b_v)
        lax.fori_loop(0, _CHUNK // 16, group, 0)
        pltpu.sync_copy(r_v, out_hbm.at[img, 0, pl.ds(r0, rows_per_chunk), :])
        pltpu.sync_copy(g_v, out_hbm.at[img, 1, pl.ds(r0, rows_per_chunk), :])
        pltpu.sync_copy(b_v, out_hbm.at[img, 2, pl.ds(r0, rows_per_chunk), :])


def _ailut_sc(imgs_flat, luts_packed, verts_pad, rcps_flat):
    mesh = plsc.VectorSubcoreMesh(core_axis_name="c", subcore_axis_name="s")
    run = functools.partial(
        pl.kernel,
        mesh=mesh,
        compiler_params=pltpu.CompilerParams(needs_layout_passes=False),
        out_type=jax.ShapeDtypeStruct((_BATCH, 3, _IMG, _IMG), jnp.float32),
        scratch_types=[
            pltpu.VMEM((_PLUT,), jnp.float32),
            pltpu.VMEM((_VERT_PAD,), jnp.float32),
            pltpu.VMEM((_RCP_PAD,), jnp.float32),
            pltpu.VMEM((_CHUNK // _IMG, _IMG), jnp.float32),
            pltpu.VMEM((_CHUNK // _IMG, _IMG), jnp.float32),
            pltpu.VMEM((_CHUNK // _IMG, _IMG), jnp.float32),
        ],
    )(_ailut_body)
    return run(imgs_flat, luts_packed, verts_pad, rcps_flat)


def _conv(x, w, b, stride):
    y = lax.conv_general_dilated(x, w, (stride, stride),
                                 padding=((1, 1), (1, 1)),
                                 dimension_numbers=('NCHW', 'OIHW', 'NCHW'))
    return y + b[None, :, None, None]


def _inorm(x, g, b, eps=1e-5):
    m = x.mean(axis=(2, 3), keepdims=True)
    v = ((x - m) ** 2).mean(axis=(2, 3), keepdims=True)
    return g[None, :, None, None] * (x - m) / jnp.sqrt(v + eps) + b[None, :, None, None]


def kernel(imgs, conv_w0, conv_b0, conv_w1, conv_b1, conv_w2, conv_b2,
           conv_w3, conv_b3, conv_w4, conv_b4, in_g0, in_b0, in_g1, in_b1,
           in_g2, in_b2, in_g3, in_b3, Wg, bg, Wl, Wa, ba):
    b = imgs.shape[0]
    x = jax.image.resize(imgs, (b, 3, 256, 256), method='bilinear')
    convs = ((conv_w0, conv_b0), (conv_w1, conv_b1), (conv_w2, conv_b2),
             (conv_w3, conv_b3), (conv_w4, conv_b4))
    norms = ((in_g0, in_b0), (in_g1, in_b1), (in_g2, in_b2), (in_g3, in_b3))
    for i in range(5):
        x = _conv(x, convs[i][0], convs[i][1], 2)
        x = jnp.maximum(x, 0.2 * x)
        if i < 4:
            x = _inorm(x, norms[i][0], norms[i][1])
    bb, cc, hh, ww = x.shape
    x = x.reshape(bb, cc, 2, hh // 2, 2, ww // 2).mean(axis=(3, 5))
    codes = x.reshape(bb, -1)

    weights = codes @ Wg.T + bg
    luts = weights @ Wl.T
    intervals = (codes @ Wa.T + ba).reshape(b, 3, _NV - 1)
    intervals = jax.nn.softmax(intervals, axis=-1)
    vertices = jnp.pad(jnp.cumsum(intervals, axis=-1), ((0, 0), (0, 0), (1, 0)))

    luts_packed = jnp.pad(luts, ((0, 0), (0, 5))).reshape(-1)
    verts_pad = jnp.pad(vertices.reshape(b, 3 * _NV),
                        ((0, 0), (0, _VERT_PAD - 3 * _NV))).reshape(-1)
    dv = vertices[..., 1:] - vertices[..., :-1]
    rcps_flat = (1.0 / (dv + 1e-10)).reshape(-1)
    return _ailut_sc(imgs, luts_packed, verts_pad, rcps_flat)
